# Initial kernel scaffold; baseline (speedup 1.0000x reference)
import jax
import jax.numpy as jnp
from jax.experimental import pallas as pl


def kernel(X, v_idx, e_idx, theta_w, theta_b, atten_e_w):
    raise NotImplementedError("write your pallas kernel here")


# speedup vs baseline: 41.5588x; 41.5588x over previous
import functools

import jax
import jax.numpy as jnp
from jax import lax
from jax.experimental import pallas as pl
from jax.experimental.pallas import tpu as pltpu
from jax.experimental.pallas import tpu_sc as plsc

N_EDGES = 5000
WIDTH = 144
MP = 5120
NP_PAD = 10240
CH = 80


def _k1_body(x_ref, w_ref, b_ref, o_ref):
    xt = jnp.dot(x_ref[...], w_ref[...],
                 preferred_element_type=jnp.float32,
                 precision=lax.Precision.HIGHEST) + b_ref[...]
    ones = jnp.ones((x_ref.shape[0], WIDTH - 128), jnp.float32)
    o_ref[...] = jnp.concatenate([xt, ones], axis=1)


def _k1(X, W, b):
    n = X.shape[0]
    bn = 1000
    return pl.pallas_call(
        _k1_body,
        grid=(n // bn,),
        in_specs=[
            pl.BlockSpec((bn, 128), lambda i: (i, 0)),
            pl.BlockSpec((128, 128), lambda i: (0, 0)),
            pl.BlockSpec((1, 128), lambda i: (0, 0)),
        ],
        out_specs=pl.BlockSpec((bn, WIDTH), lambda i: (i, 0)),
        out_shape=jax.ShapeDtypeStruct((n, WIDTH), jnp.float32),
    )(X, W, b)


def _k3_body(acc_ref, a_ref, s_ref, o_ref):
    tot = acc_ref[0] + acc_ref[1]
    cnt = jnp.maximum(tot[:, 128:129], 1.0)
    y = tot[:, :128] / cnt
    alpha = jnp.dot(y, a_ref[...],
                    preferred_element_type=jnp.float32,
                    precision=lax.Precision.HIGHEST)
    lr = jnp.where(alpha >= 0.0, alpha, 0.2 * alpha)
    g = jnp.maximum(jnp.max(lr, axis=0, keepdims=True), 0.0)
    e = jnp.exp(lr - g)
    ew = jnp.dot(e, s_ref[...],
                 preferred_element_type=jnp.float32,
                 precision=lax.Precision.HIGHEST)
    o_ref[...] = jnp.concatenate(
        [y * ew, e, jnp.zeros((y.shape[0], 8), jnp.float32)], axis=1)


def _k3(acc, A, S):
    return pl.pallas_call(
        _k3_body,
        out_shape=jax.ShapeDtypeStruct((MP, WIDTH), jnp.float32),
    )(acc, A, S)


def _k5_body(acc_ref, s_ref, o_ref):
    tot = acc_ref[0] + acc_ref[1]
    numer = tot[:, :128]
    den = jnp.dot(tot[:, 128:136], s_ref[...],
                  preferred_element_type=jnp.float32,
                  precision=lax.Precision.HIGHEST) + 1e-12
    r = numer / den
    o_ref[...] = jnp.where(r > 0.0, r, jnp.exp(jnp.minimum(r, 0.0)) - 1.0)


def _k5(acc, S):
    bn = 1024
    return pl.pallas_call(
        _k5_body,
        grid=(NP_PAD // bn,),
        in_specs=[
            pl.BlockSpec((2, bn, WIDTH), lambda i: (0, i, 0)),
            pl.BlockSpec((8, 128), lambda i: (0, 0)),
        ],
        out_specs=pl.BlockSpec((bn, 128), lambda i: (i, 0)),
        out_shape=jax.ShapeDtypeStruct((NP_PAD, 128), jnp.float32),
    )(acc, S)


def _sc_pass(table, sidx, didx, zeros, n_dst_pad):
    p = sidx.shape[0]
    n_chunks = p // (32 * CH)
    rpt = n_dst_pad // 16
    mesh = plsc.VectorSubcoreMesh(core_axis_name="c", subcore_axis_name="s")

    @functools.partial(
        pl.kernel, mesh=mesh,
        compiler_params=pltpu.CompilerParams(use_tc_tiling_on_sc=False),
        out_type=jax.ShapeDtypeStruct((2, n_dst_pad, WIDTH), jnp.float32),
        scratch_types=[
            pltpu.VMEM((CH,), jnp.int32),
            pltpu.VMEM((CH,), jnp.int32),
            pltpu.VMEM((CH, WIDTH), jnp.float32),
            pltpu.VMEM_SHARED((n_dst_pad, WIDTH), jnp.float32),
            pltpu.SemaphoreType.DMA,
        ],
    )
    def k(table_hbm, sidx_hbm, didx_hbm, z_hbm, out_hbm,
          sbuf, dbuf, rows, acc, sem):
        c = lax.axis_index("c")
        s = lax.axis_index("s")
        wid = s * 2 + c
        base = wid * (p // 32)
        pltpu.sync_copy(z_hbm.at[pl.ds(s * rpt, rpt)],
                        acc.at[pl.ds(s * rpt, rpt)])
        plsc.subcore_barrier()

        def body(i, carry):
            off = base + i * CH
            pltpu.sync_copy(sidx_hbm.at[pl.ds(off, CH)], sbuf)
            pltpu.sync_copy(didx_hbm.at[pl.ds(off, CH)], dbuf)
            pltpu.async_copy(table_hbm.at[sbuf], rows, sem).wait()
            pltpu.sync_copy(rows, acc.at[dbuf], add=True)
            return carry

        lax.fori_loop(0, n_chunks, body, 0)
        plsc.subcore_barrier()
        pltpu.sync_copy(acc.at[pl.ds(s * rpt, rpt)],
                        out_hbm.at[c, pl.ds(s * rpt, rpt)])

    return k(table, sidx, didx, zeros)


def kernel(X, v_idx, e_idx, theta_w, theta_b, atten_e_w):
    n, in_c = X.shape
    h, _, oc = theta_w.shape
    hid = h * oc
    w_all = jnp.transpose(theta_w, (1, 0, 2)).reshape(in_c, hid)
    b_all = theta_b.reshape(1, hid)
    flat = atten_e_w.reshape(hid)
    mask = (jnp.arange(hid)[:, None] // oc
            == jnp.arange(8)[None, :]).astype(jnp.float32)
    a_mat = flat[:, None] * mask
    s_mat = mask.T
    v32 = v_idx.astype(jnp.int32)
    e32 = e_idx.astype(jnp.int32)

    t1 = _k1(X, w_all, b_all)
    acc1 = _sc_pass(t1, v32, e32,
                    jnp.zeros((MP, WIDTH), jnp.float32), MP)
    t2 = _k3(acc1, a_mat, s_mat)
    acc2 = _sc_pass(t2, e32, v32,
                    jnp.zeros((NP_PAD, WIDTH), jnp.float32), NP_PAD)
    out = _k5(acc2, s_mat)
    return out[:n]


# speedup vs baseline: 67.4921x; 1.6240x over previous
import functools

import jax
import jax.numpy as jnp
from jax import lax
from jax.experimental import pallas as pl
from jax.experimental.pallas import tpu as pltpu
from jax.experimental.pallas import tpu_sc as plsc

N_EDGES = 5000
WIDTH = 144
MP = 5008
NP_PAD = 10000
CH = 40


def _k1_body(x_ref, w_ref, b_ref, o_ref):
    xt = jnp.dot(x_ref[...], w_ref[...],
                 preferred_element_type=jnp.float32,
                 precision=lax.Precision.HIGHEST) + b_ref[...]
    ones = jnp.ones((x_ref.shape[0], WIDTH - 128), jnp.float32)
    o_ref[...] = jnp.concatenate([xt, ones], axis=1)


def _k1(X, W, b):
    n = X.shape[0]
    bn = 1000
    return pl.pallas_call(
        _k1_body,
        grid=(n // bn,),
        in_specs=[
            pl.BlockSpec((bn, 128), lambda i: (i, 0)),
            pl.BlockSpec((128, 128), lambda i: (0, 0)),
            pl.BlockSpec((1, 128), lambda i: (0, 0)),
        ],
        out_specs=pl.BlockSpec((bn, WIDTH), lambda i: (i, 0)),
        out_shape=jax.ShapeDtypeStruct((n, WIDTH), jnp.float32),
    )(X, W, b)


def _k3_body(acc_ref, a_ref, s_ref, o_ref):
    tot = acc_ref[0] + acc_ref[1]
    cnt = jnp.maximum(tot[:, 128:129], 1.0)
    y = tot[:, :128] / cnt
    alpha = jnp.dot(y, a_ref[...],
                    preferred_element_type=jnp.float32,
                    precision=lax.Precision.HIGHEST)
    lr = jnp.where(alpha >= 0.0, alpha, 0.2 * alpha)
    g = jnp.maximum(jnp.max(lr, axis=0, keepdims=True), 0.0)
    e = jnp.exp(lr - g)
    ew = jnp.dot(e, s_ref[...],
                 preferred_element_type=jnp.float32,
                 precision=lax.Precision.HIGHEST)
    o_ref[...] = jnp.concatenate(
        [y * ew, e, jnp.zeros((y.shape[0], 8), jnp.float32)], axis=1)


def _k3(acc, A, S):
    return pl.pallas_call(
        _k3_body,
        out_shape=jax.ShapeDtypeStruct((MP, WIDTH), jnp.float32),
    )(acc, A, S)


def _k5_body(acc_ref, s_ref, o_ref):
    tot = acc_ref[0] + acc_ref[1]
    numer = tot[:, :128]
    den = jnp.dot(tot[:, 128:136], s_ref[...],
                  preferred_element_type=jnp.float32,
                  precision=lax.Precision.HIGHEST) + 1e-12
    r = numer / den
    o_ref[...] = jnp.where(r > 0.0, r, jnp.exp(jnp.minimum(r, 0.0)) - 1.0)


def _k5(acc, S):
    bn = 1000
    return pl.pallas_call(
        _k5_body,
        grid=(NP_PAD // bn,),
        in_specs=[
            pl.BlockSpec((2, bn, WIDTH), lambda i: (0, i, 0)),
            pl.BlockSpec((8, 128), lambda i: (0, 0)),
        ],
        out_specs=pl.BlockSpec((bn, 128), lambda i: (i, 0)),
        out_shape=jax.ShapeDtypeStruct((NP_PAD, 128), jnp.float32),
    )(acc, S)


def _sc_pass(table, sidx, didx, zeros, n_dst_pad):
    n_chunks = sidx.shape[1]
    rpt = n_dst_pad // 16
    mesh = plsc.VectorSubcoreMesh(core_axis_name="c", subcore_axis_name="s")

    @functools.partial(
        pl.kernel, mesh=mesh,
        compiler_params=pltpu.CompilerParams(use_tc_tiling_on_sc=False),
        out_type=jax.ShapeDtypeStruct((2, n_dst_pad, WIDTH), jnp.float32),
        scratch_types=[
            pltpu.VMEM((n_chunks, CH), jnp.int32),
            pltpu.VMEM((n_chunks, CH), jnp.int32),
            pltpu.VMEM((2, CH, WIDTH), jnp.float32),
            pltpu.VMEM_SHARED((n_dst_pad, WIDTH), jnp.float32),
            pltpu.SemaphoreType.DMA,
        ],
    )
    def k(table_hbm, sidx_hbm, didx_hbm, z_hbm, out_hbm,
          sbuf, dbuf, rows, acc, sem):
        c = lax.axis_index("c")
        s = lax.axis_index("s")
        wid = s * 2 + c
        pltpu.sync_copy(sidx_hbm.at[wid], sbuf)
        pltpu.sync_copy(didx_hbm.at[wid], dbuf)
        pltpu.sync_copy(z_hbm.at[pl.ds(s * rpt, rpt)],
                        acc.at[pl.ds(s * rpt, rpt)])
        plsc.subcore_barrier()

        pltpu.async_copy(table_hbm.at[sbuf.at[0]], rows.at[0], sem)

        def body(i, carry):
            b = lax.rem(i, 2)

            @pl.when(i + 1 < n_chunks)
            def _():
                pltpu.async_copy(table_hbm.at[sbuf.at[i + 1]],
                                 rows.at[1 - b], sem)

            pltpu.make_async_copy(table_hbm.at[sbuf.at[i]],
                                  rows.at[b], sem).wait()
            pltpu.sync_copy(rows.at[b], acc.at[dbuf.at[i]], add=True)
            return carry

        lax.fori_loop(0, n_chunks, body, 0)
        plsc.subcore_barrier()
        pltpu.sync_copy(acc.at[pl.ds(s * rpt, rpt)],
                        out_hbm.at[c, pl.ds(s * rpt, rpt)])

    return k(table, sidx, didx, zeros)


def kernel(X, v_idx, e_idx, theta_w, theta_b, atten_e_w):
    n, in_c = X.shape
    h, _, oc = theta_w.shape
    hid = h * oc
    w_all = jnp.transpose(theta_w, (1, 0, 2)).reshape(in_c, hid)
    b_all = theta_b.reshape(1, hid)
    flat = atten_e_w.reshape(hid)
    mask = (jnp.arange(hid)[:, None] // oc
            == jnp.arange(8)[None, :]).astype(jnp.float32)
    a_mat = flat[:, None] * mask
    s_mat = mask.T
    p = v_idx.shape[0]
    n_chunks = p // (32 * CH)
    v32 = v_idx.astype(jnp.int32).reshape(32, n_chunks, CH)
    e32 = e_idx.astype(jnp.int32).reshape(32, n_chunks, CH)

    t1 = _k1(X, w_all, b_all)
    acc1 = _sc_pass(t1, v32, e32,
                    jnp.zeros((MP, WIDTH), jnp.float32), MP)
    t2 = _k3(acc1, a_mat, s_mat)
    acc2 = _sc_pass(t2, e32, v32,
                    jnp.zeros((NP_PAD, WIDTH), jnp.float32), NP_PAD)
    out = _k5(acc2, s_mat)
    return out[:n]


# speedup vs baseline: 70.4772x; 1.0442x over previous
import functools

import jax
import jax.numpy as jnp
from jax import lax
from jax.experimental import pallas as pl
from jax.experimental.pallas import tpu as pltpu
from jax.experimental.pallas import tpu_sc as plsc

N_EDGES = 5000
WIDTH = 144
MP = 5008
NP_PAD = 10000
CH = 40


def _k1_body(x_ref, w_ref, b_ref, o_ref):
    xt = jnp.dot(x_ref[...], w_ref[...],
                 preferred_element_type=jnp.float32,
                 precision=lax.Precision.HIGHEST) + b_ref[...]
    ones = jnp.ones((x_ref.shape[0], WIDTH - 128), jnp.float32)
    o_ref[...] = jnp.concatenate([xt, ones], axis=1)


def _k1(X, W, b):
    n = X.shape[0]
    bn = 1000
    return pl.pallas_call(
        _k1_body,
        grid=(n // bn,),
        in_specs=[
            pl.BlockSpec((bn, 128), lambda i: (i, 0)),
            pl.BlockSpec((128, 128), lambda i: (0, 0)),
            pl.BlockSpec((1, 128), lambda i: (0, 0)),
        ],
        out_specs=pl.BlockSpec((bn, WIDTH), lambda i: (i, 0)),
        out_shape=jax.ShapeDtypeStruct((n, WIDTH), jnp.float32),
    )(X, W, b)


def _k3_body(acc_ref, a_ref, s_ref, o_ref):
    tot = acc_ref[0] + acc_ref[1]
    cnt = jnp.maximum(tot[:, 128:129], 1.0)
    y = tot[:, :128] / cnt
    alpha = jnp.dot(y, a_ref[...],
                    preferred_element_type=jnp.float32,
                    precision=lax.Precision.HIGHEST)
    lr = jnp.where(alpha >= 0.0, alpha, 0.2 * alpha)
    g = jnp.maximum(jnp.max(lr, axis=0, keepdims=True), 0.0)
    e = jnp.exp(lr - g)
    ew = jnp.dot(e, s_ref[...],
                 preferred_element_type=jnp.float32,
                 precision=lax.Precision.HIGHEST)
    o_ref[...] = jnp.concatenate(
        [y * ew, e, jnp.zeros((y.shape[0], 8), jnp.float32)], axis=1)


def _k3(acc, A, S):
    return pl.pallas_call(
        _k3_body,
        out_shape=jax.ShapeDtypeStruct((MP, WIDTH), jnp.float32),
    )(acc, A, S)


def _k5_body(acc_ref, s_ref, o_ref):
    tot = acc_ref[0] + acc_ref[1]
    numer = tot[:, :128]
    den = jnp.dot(tot[:, 128:136], s_ref[...],
                  preferred_element_type=jnp.float32,
                  precision=lax.Precision.HIGHEST) + 1e-12
    r = numer / den
    o_ref[...] = jnp.where(r > 0.0, r, jnp.exp(jnp.minimum(r, 0.0)) - 1.0)


def _k5(acc, S):
    bn = 1000
    return pl.pallas_call(
        _k5_body,
        grid=(NP_PAD // bn,),
        in_specs=[
            pl.BlockSpec((2, bn, WIDTH), lambda i: (0, i, 0)),
            pl.BlockSpec((8, 128), lambda i: (0, 0)),
        ],
        out_specs=pl.BlockSpec((bn, 128), lambda i: (i, 0)),
        out_shape=jax.ShapeDtypeStruct((NP_PAD, 128), jnp.float32),
    )(acc, S)


def _sc_pass(table, sidx, didx, zeros, n_dst_pad):
    n_chunks = sidx.shape[1]
    rpt = n_dst_pad // 16
    mesh = plsc.VectorSubcoreMesh(core_axis_name="c", subcore_axis_name="s")

    @functools.partial(
        pl.kernel, mesh=mesh,
        compiler_params=pltpu.CompilerParams(use_tc_tiling_on_sc=False),
        out_type=jax.ShapeDtypeStruct((2, n_dst_pad, WIDTH), jnp.float32),
        scratch_types=[
            pltpu.VMEM((n_chunks, CH), jnp.int32),
            pltpu.VMEM((n_chunks, CH), jnp.int32),
            pltpu.VMEM((3, CH, WIDTH), jnp.float32),
            pltpu.VMEM_SHARED((n_dst_pad, WIDTH), jnp.float32),
            pltpu.SemaphoreType.DMA,
            pltpu.SemaphoreType.DMA,
        ],
    )
    def k(table_hbm, sidx_hbm, didx_hbm, z_hbm, out_hbm,
          sbuf, dbuf, rows, acc, sem_g, sem_s):
        c = lax.axis_index("c")
        s = lax.axis_index("s")
        wid = s * 2 + c
        pltpu.sync_copy(sidx_hbm.at[wid], sbuf)
        pltpu.sync_copy(didx_hbm.at[wid], dbuf)
        pltpu.sync_copy(z_hbm.at[pl.ds(s * rpt, rpt)],
                        acc.at[pl.ds(s * rpt, rpt)])
        plsc.subcore_barrier()

        pltpu.async_copy(table_hbm.at[sbuf.at[0]], rows.at[0], sem_g)
        pltpu.async_copy(table_hbm.at[sbuf.at[1]], rows.at[1], sem_g)

        def body(i, carry):
            b = lax.rem(i, 3)
            pltpu.make_async_copy(table_hbm.at[sbuf.at[i]],
                                  rows.at[b], sem_g).wait()

            @pl.when(i >= 1)
            def _():
                pltpu.make_async_copy(rows.at[b], acc.at[dbuf.at[i]],
                                      sem_s).wait()

            pltpu.async_copy(rows.at[b], acc.at[dbuf.at[i]], sem_s,
                             add=True)

            @pl.when(i + 2 < n_chunks)
            def _():
                pltpu.async_copy(table_hbm.at[sbuf.at[i + 2]],
                                 rows.at[lax.rem(i + 2, 3)], sem_g)

            return carry

        lax.fori_loop(0, n_chunks, body, 0)
        pltpu.make_async_copy(rows.at[0], acc.at[dbuf.at[0]], sem_s).wait()
        plsc.subcore_barrier()
        pltpu.sync_copy(acc.at[pl.ds(s * rpt, rpt)],
                        out_hbm.at[c, pl.ds(s * rpt, rpt)])

    return k(table, sidx, didx, zeros)


def kernel(X, v_idx, e_idx, theta_w, theta_b, atten_e_w):
    n, in_c = X.shape
    h, _, oc = theta_w.shape
    hid = h * oc
    w_all = jnp.transpose(theta_w, (1, 0, 2)).reshape(in_c, hid)
    b_all = theta_b.reshape(1, hid)
    flat = atten_e_w.reshape(hid)
    mask = (jnp.arange(hid)[:, None] // oc
            == jnp.arange(8)[None, :]).astype(jnp.float32)
    a_mat = flat[:, None] * mask
    s_mat = mask.T
    p = v_idx.shape[0]
    n_chunks = p // (32 * CH)
    v32 = v_idx.astype(jnp.int32).reshape(32, n_chunks, CH)
    e32 = e_idx.astype(jnp.int32).reshape(32, n_chunks, CH)

    t1 = _k1(X, w_all, b_all)
    acc1 = _sc_pass(t1, v32, e32,
                    jnp.zeros((MP, WIDTH), jnp.float32), MP)
    t2 = _k3(acc1, a_mat, s_mat)
    acc2 = _sc_pass(t2, e32, v32,
                    jnp.zeros((NP_PAD, WIDTH), jnp.float32), NP_PAD)
    out = _k5(acc2, s_mat)
    return out[:n]


# speedup vs baseline: 80.1908x; 1.1378x over previous
import functools

import jax
import jax.numpy as jnp
from jax import lax
from jax.experimental import pallas as pl
from jax.experimental.pallas import tpu as pltpu
from jax.experimental.pallas import tpu_sc as plsc

N_EDGES = 5000
WIDTH = 144
MP = 5008
NP_PAD = 10000
CH = 40


def _k1_body(x_ref, w_ref, b_ref, o_ref):
    xt = jnp.dot(x_ref[...], w_ref[...],
                 preferred_element_type=jnp.float32,
                 precision=lax.Precision.HIGHEST) + b_ref[...]
    ones = jnp.ones((x_ref.shape[0], WIDTH - 128), jnp.float32)
    o_ref[...] = jnp.concatenate([xt, ones], axis=1)


def _k1(X, W, b):
    n = X.shape[0]
    bn = 1000
    return pl.pallas_call(
        _k1_body,
        grid=(n // bn,),
        in_specs=[
            pl.BlockSpec((bn, 128), lambda i: (i, 0)),
            pl.BlockSpec((128, 128), lambda i: (0, 0)),
            pl.BlockSpec((1, 128), lambda i: (0, 0)),
        ],
        out_specs=pl.BlockSpec((bn, WIDTH), lambda i: (i, 0)),
        out_shape=jax.ShapeDtypeStruct((n, WIDTH), jnp.float32),
    )(X, W, b)


def _k3_body(acc_ref, a_ref, s_ref, o_ref):
    tot = acc_ref[0] + acc_ref[1]
    cnt = jnp.maximum(tot[:, 128:129], 1.0)
    y = tot[:, :128] / cnt
    alpha = jnp.dot(y, a_ref[...],
                    preferred_element_type=jnp.float32,
                    precision=lax.Precision.HIGHEST)
    lr = jnp.where(alpha >= 0.0, alpha, 0.2 * alpha)
    g = jnp.maximum(jnp.max(lr, axis=0, keepdims=True), 0.0)
    e = jnp.exp(lr - g)
    ew = jnp.dot(e, s_ref[...],
                 preferred_element_type=jnp.float32,
                 precision=lax.Precision.HIGHEST)
    o_ref[...] = jnp.concatenate(
        [y * ew, e, jnp.zeros((y.shape[0], 8), jnp.float32)], axis=1)


def _k3(acc, A, S):
    return pl.pallas_call(
        _k3_body,
        out_shape=jax.ShapeDtypeStruct((MP, WIDTH), jnp.float32),
    )(acc, A, S)


def _k5_body(acc_ref, s_ref, o_ref):
    tot = acc_ref[0] + acc_ref[1]
    numer = tot[:, :128]
    den = jnp.dot(tot[:, 128:136], s_ref[...],
                  preferred_element_type=jnp.float32,
                  precision=lax.Precision.HIGHEST) + 1e-12
    r = numer / den
    o_ref[...] = jnp.where(r > 0.0, r, jnp.exp(jnp.minimum(r, 0.0)) - 1.0)


def _k5(acc, S):
    bn = 1000
    return pl.pallas_call(
        _k5_body,
        grid=(NP_PAD // bn,),
        in_specs=[
            pl.BlockSpec((2, bn, WIDTH), lambda i: (0, i, 0)),
            pl.BlockSpec((8, 128), lambda i: (0, 0)),
        ],
        out_specs=pl.BlockSpec((bn, 128), lambda i: (i, 0)),
        out_shape=jax.ShapeDtypeStruct((NP_PAD, 128), jnp.float32),
    )(acc, S)


def _sc_pass(table, sidx, didx, zeros, n_dst_pad, ch):
    n_chunks = sidx.shape[1]
    assert sidx.shape[2] == ch
    rpt = n_dst_pad // 16
    mesh = plsc.VectorSubcoreMesh(core_axis_name="c", subcore_axis_name="s")

    @functools.partial(
        pl.kernel, mesh=mesh,
        compiler_params=pltpu.CompilerParams(use_tc_tiling_on_sc=False),
        out_type=jax.ShapeDtypeStruct((2, n_dst_pad, WIDTH), jnp.float32),
        scratch_types=[
            pltpu.VMEM((n_chunks, ch), jnp.int32),
            pltpu.VMEM((n_chunks, ch), jnp.int32),
            pltpu.VMEM((3, ch, WIDTH), jnp.float32),
            pltpu.VMEM_SHARED((n_dst_pad, WIDTH), jnp.float32),
            pltpu.SemaphoreType.DMA,
            pltpu.SemaphoreType.DMA,
        ],
    )
    def k(table_hbm, sidx_hbm, didx_hbm, z_hbm, out_hbm,
          sbuf, dbuf, rows, acc, sem_g, sem_s):
        c = lax.axis_index("c")
        s = lax.axis_index("s")
        wid = s * 2 + c
        pltpu.sync_copy(sidx_hbm.at[wid], sbuf)
        pltpu.sync_copy(didx_hbm.at[wid], dbuf)
        pltpu.sync_copy(z_hbm.at[pl.ds(s * rpt, rpt)],
                        acc.at[pl.ds(s * rpt, rpt)])
        plsc.subcore_barrier()

        pltpu.async_copy(table_hbm.at[sbuf.at[0]], rows.at[0], sem_g)
        pltpu.async_copy(table_hbm.at[sbuf.at[1]], rows.at[1], sem_g)

        def body(i, carry):
            b = lax.rem(i, 3)
            pltpu.make_async_copy(table_hbm.at[sbuf.at[i]],
                                  rows.at[b], sem_g).wait()

            @pl.when(i >= 1)
            def _():
                pltpu.make_async_copy(rows.at[b], acc.at[dbuf.at[i]],
                                      sem_s).wait()

            pltpu.async_copy(rows.at[b], acc.at[dbuf.at[i]], sem_s,
                             add=True)

            @pl.when(i + 2 < n_chunks)
            def _():
                pltpu.async_copy(table_hbm.at[sbuf.at[i + 2]],
                                 rows.at[lax.rem(i + 2, 3)], sem_g)

            return carry

        lax.fori_loop(0, n_chunks, body, 0)
        pltpu.make_async_copy(rows.at[0], acc.at[dbuf.at[0]], sem_s).wait()
        plsc.subcore_barrier()
        pltpu.sync_copy(acc.at[pl.ds(s * rpt, rpt)],
                        out_hbm.at[c, pl.ds(s * rpt, rpt)])

    return k(table, sidx, didx, zeros)


def kernel(X, v_idx, e_idx, theta_w, theta_b, atten_e_w):
    n, in_c = X.shape
    h, _, oc = theta_w.shape
    hid = h * oc
    w_all = jnp.transpose(theta_w, (1, 0, 2)).reshape(in_c, hid)
    b_all = theta_b.reshape(1, hid)
    flat = atten_e_w.reshape(hid)
    mask = (jnp.arange(hid)[:, None] // oc
            == jnp.arange(8)[None, :]).astype(jnp.float32)
    a_mat = flat[:, None] * mask
    s_mat = mask.T
    p = v_idx.shape[0]
    ch1, ch2 = 80, CH
    v32 = v_idx.astype(jnp.int32)
    e32 = e_idx.astype(jnp.int32)

    t1 = _k1(X, w_all, b_all)
    acc1 = _sc_pass(t1,
                    v32.reshape(32, p // (32 * ch1), ch1),
                    e32.reshape(32, p // (32 * ch1), ch1),
                    jnp.zeros((MP, WIDTH), jnp.float32), MP, ch1)
    t2 = _k3(acc1, a_mat, s_mat)
    acc2 = _sc_pass(t2,
                    e32.reshape(32, p // (32 * ch2), ch2),
                    v32.reshape(32, p // (32 * ch2), ch2),
                    jnp.zeros((NP_PAD, WIDTH), jnp.float32), NP_PAD, ch2)
    out = _k5(acc2, s_mat)
    return out[:n]


# speedup vs baseline: 82.9563x; 1.0345x over previous
import functools

import jax
import jax.numpy as jnp
from jax import lax
from jax.experimental import pallas as pl
from jax.experimental.pallas import tpu as pltpu
from jax.experimental.pallas import tpu_sc as plsc

N_EDGES = 5000
WIDTH = 144
MP = 5008
NP_PAD = 10000
CH = 80


def _k1_body(x_ref, w_ref, b_ref, o_ref):
    xt = jnp.dot(x_ref[...], w_ref[...],
                 preferred_element_type=jnp.float32,
                 precision=lax.Precision.HIGHEST) + b_ref[...]
    ones = jnp.ones((x_ref.shape[0], WIDTH - 128), jnp.float32)
    o_ref[...] = jnp.concatenate([xt, ones], axis=1)


def _k1(X, W, b):
    n = X.shape[0]
    bn = 1000
    return pl.pallas_call(
        _k1_body,
        grid=(n // bn,),
        in_specs=[
            pl.BlockSpec((bn, 128), lambda i: (i, 0)),
            pl.BlockSpec((128, 128), lambda i: (0, 0)),
            pl.BlockSpec((1, 128), lambda i: (0, 0)),
        ],
        out_specs=pl.BlockSpec((bn, WIDTH), lambda i: (i, 0)),
        out_shape=jax.ShapeDtypeStruct((n, WIDTH), jnp.float32),
    )(X, W, b)


def _k3_body(acc_ref, a_ref, s_ref, o_ref):
    tot = acc_ref[0] + acc_ref[1]
    cnt = jnp.maximum(tot[:, 128:129], 1.0)
    y = tot[:, :128] / cnt
    alpha = jnp.dot(y, a_ref[...],
                    preferred_element_type=jnp.float32,
                    precision=lax.Precision.HIGHEST)
    lr = jnp.where(alpha >= 0.0, alpha, 0.2 * alpha)
    g = jnp.maximum(jnp.max(lr, axis=0, keepdims=True), 0.0)
    e = jnp.exp(lr - g)
    ew = jnp.dot(e, s_ref[...],
                 preferred_element_type=jnp.float32,
                 precision=lax.Precision.HIGHEST)
    o_ref[...] = jnp.concatenate(
        [y * ew, e, jnp.zeros((y.shape[0], 8), jnp.float32)], axis=1)


def _k3(acc, A, S):
    return pl.pallas_call(
        _k3_body,
        out_shape=jax.ShapeDtypeStruct((MP, WIDTH), jnp.float32),
    )(acc, A, S)


def _k5_body(acc_ref, s_ref, o_ref):
    tot = acc_ref[0] + acc_ref[1]
    numer = tot[:, :128]
    den = jnp.dot(tot[:, 128:136], s_ref[...],
                  preferred_element_type=jnp.float32,
                  precision=lax.Precision.HIGHEST) + 1e-12
    r = numer / den
    o_ref[...] = jnp.where(r > 0.0, r, jnp.exp(jnp.minimum(r, 0.0)) - 1.0)


def _k5(acc, S):
    bn = 1000
    return pl.pallas_call(
        _k5_body,
        grid=(NP_PAD // bn,),
        in_specs=[
            pl.BlockSpec((2, bn, WIDTH), lambda i: (0, i, 0)),
            pl.BlockSpec((8, 128), lambda i: (0, 0)),
        ],
        out_specs=pl.BlockSpec((bn, 128), lambda i: (i, 0)),
        out_shape=jax.ShapeDtypeStruct((NP_PAD, 128), jnp.float32),
    )(acc, S)


def _sc_pass(table, idx, zeros, n_dst_pad, ch):
    n_chunks = idx.shape[1]
    rpt = n_dst_pad // 16
    mesh = plsc.VectorSubcoreMesh(core_axis_name="c", subcore_axis_name="s")

    @functools.partial(
        pl.kernel, mesh=mesh,
        compiler_params=pltpu.CompilerParams(use_tc_tiling_on_sc=False),
        out_type=jax.ShapeDtypeStruct((2, n_dst_pad, WIDTH), jnp.float32),
        scratch_types=[
            pltpu.VMEM((4, 2, ch), jnp.int32),
            pltpu.VMEM((3, ch, WIDTH), jnp.float32),
            pltpu.VMEM_SHARED((n_dst_pad, WIDTH), jnp.float32),
            pltpu.SemaphoreType.DMA,
            pltpu.SemaphoreType.DMA,
            pltpu.SemaphoreType.DMA,
        ],
    )
    def k(table_hbm, idx_hbm, z_hbm, out_hbm,
          ibuf, rows, acc, sem_i, sem_g, sem_s):
        c = lax.axis_index("c")
        s = lax.axis_index("s")
        wid = s * 2 + c
        for j in range(3):
            pltpu.async_copy(idx_hbm.at[wid, j], ibuf.at[j], sem_i)
        pltpu.sync_copy(z_hbm.at[pl.ds(s * rpt, rpt)],
                        acc.at[pl.ds(s * rpt, rpt)])
        plsc.subcore_barrier()

        def wait_idx():
            pltpu.make_async_copy(idx_hbm.at[wid, 0], ibuf.at[0],
                                  sem_i).wait()

        wait_idx()
        pltpu.async_copy(table_hbm.at[ibuf.at[0, 0]], rows.at[0], sem_g)
        wait_idx()
        pltpu.async_copy(table_hbm.at[ibuf.at[1, 0]], rows.at[1], sem_g)

        def body(i, carry):
            b = lax.rem(i, 3)
            bi = lax.rem(i, 4)
            pltpu.make_async_copy(table_hbm.at[ibuf.at[bi, 0]],
                                  rows.at[b], sem_g).wait()

            @pl.when(i >= 1)
            def _():
                pltpu.make_async_copy(rows.at[b], acc.at[ibuf.at[bi, 1]],
                                      sem_s).wait()

            pltpu.async_copy(rows.at[b], acc.at[ibuf.at[bi, 1]], sem_s,
                             add=True)

            @pl.when(i + 2 < n_chunks)
            def _():
                wait_idx()
                pltpu.async_copy(table_hbm.at[ibuf.at[lax.rem(i + 2, 4), 0]],
                                 rows.at[lax.rem(i + 2, 3)], sem_g)

            @pl.when(i + 3 < n_chunks)
            def _():
                pltpu.async_copy(idx_hbm.at[wid, i + 3],
                                 ibuf.at[lax.rem(i + 3, 4)], sem_i)

            return carry

        lax.fori_loop(0, n_chunks, body, 0)
        pltpu.make_async_copy(rows.at[0], acc.at[ibuf.at[0, 1]], sem_s).wait()
        plsc.subcore_barrier()
        pltpu.sync_copy(acc.at[pl.ds(s * rpt, rpt)],
                        out_hbm.at[c, pl.ds(s * rpt, rpt)])

    return k(table, idx, zeros)


def kernel(X, v_idx, e_idx, theta_w, theta_b, atten_e_w):
    n, in_c = X.shape
    h, _, oc = theta_w.shape
    hid = h * oc
    w_all = jnp.transpose(theta_w, (1, 0, 2)).reshape(in_c, hid)
    b_all = theta_b.reshape(1, hid)
    flat = atten_e_w.reshape(hid)
    mask = (jnp.arange(hid)[:, None] // oc
            == jnp.arange(8)[None, :]).astype(jnp.float32)
    a_mat = flat[:, None] * mask
    s_mat = mask.T
    p = v_idx.shape[0]
    nc = p // (32 * CH)
    v32 = v_idx.astype(jnp.int32).reshape(32, nc, 1, CH)
    e32 = e_idx.astype(jnp.int32).reshape(32, nc, 1, CH)
    idx_ve = jnp.concatenate([v32, e32], axis=2)
    idx_ev = jnp.concatenate([e32, v32], axis=2)

    t1 = _k1(X, w_all, b_all)
    acc1 = _sc_pass(t1, idx_ve,
                    jnp.zeros((MP, WIDTH), jnp.float32), MP, CH)
    t2 = _k3(acc1, a_mat, s_mat)
    acc2 = _sc_pass(t2, idx_ev,
                    jnp.zeros((NP_PAD, WIDTH), jnp.float32), NP_PAD, CH)
    out = _k5(acc2, s_mat)
    return out[:n]


# speedup vs baseline: 85.2327x; 1.0274x over previous
import functools

import jax
import jax.numpy as jnp
from jax import lax
from jax.experimental import pallas as pl
from jax.experimental.pallas import tpu as pltpu
from jax.experimental.pallas import tpu_sc as plsc

N_EDGES = 5000
WIDTH = 144
MP = 5040
NP_PAD = 10000
CH = 80


def _k1_body(x_ref, w_ref, b_ref, v_ref, e_ref,
             o_ref, zm_ref, zn_ref, ve_ref, ev_ref):
    xt = jnp.dot(x_ref[...], w_ref[...],
                 preferred_element_type=jnp.float32,
                 precision=lax.Precision.HIGHEST) + b_ref[...]
    ones = jnp.ones((x_ref.shape[0], WIDTH - 128), jnp.float32)
    o_ref[...] = jnp.concatenate([xt, ones], axis=1)
    zm_ref[...] = jnp.zeros_like(zm_ref)
    zn_ref[...] = jnp.zeros_like(zn_ref)
    v = v_ref[...]
    e = e_ref[...]
    ve_ref[...] = jnp.concatenate([v[:, None, :], e[:, None, :]], axis=1)
    ev_ref[...] = jnp.concatenate([e[:, None, :], v[:, None, :]], axis=1)


def _k1(X, W, b, vf, ef):
    n = X.shape[0]
    bn = 1000
    g = n // bn
    nr = vf.shape[0]
    return pl.pallas_call(
        _k1_body,
        grid=(g,),
        in_specs=[
            pl.BlockSpec((bn, 128), lambda i: (i, 0)),
            pl.BlockSpec((128, 128), lambda i: (0, 0)),
            pl.BlockSpec((1, 128), lambda i: (0, 0)),
            pl.BlockSpec((nr // g, CH), lambda i: (i, 0)),
            pl.BlockSpec((nr // g, CH), lambda i: (i, 0)),
        ],
        out_specs=[
            pl.BlockSpec((bn, WIDTH), lambda i: (i, 0)),
            pl.BlockSpec((MP // g, WIDTH), lambda i: (i, 0)),
            pl.BlockSpec((NP_PAD // g, WIDTH), lambda i: (i, 0)),
            pl.BlockSpec((nr // g, 2, CH), lambda i: (i, 0, 0)),
            pl.BlockSpec((nr // g, 2, CH), lambda i: (i, 0, 0)),
        ],
        out_shape=[
            jax.ShapeDtypeStruct((n, WIDTH), jnp.float32),
            jax.ShapeDtypeStruct((MP, WIDTH), jnp.float32),
            jax.ShapeDtypeStruct((NP_PAD, WIDTH), jnp.float32),
            jax.ShapeDtypeStruct((nr, 2, CH), jnp.int32),
            jax.ShapeDtypeStruct((nr, 2, CH), jnp.int32),
        ],
    )(X, W, b, vf, ef)


def _k3_body(acc_ref, a_ref, s_ref, o_ref):
    tot = acc_ref[0] + acc_ref[1]
    cnt = jnp.maximum(tot[:, 128:129], 1.0)
    y = tot[:, :128] / cnt
    alpha = jnp.dot(y, a_ref[...],
                    preferred_element_type=jnp.float32,
                    precision=lax.Precision.HIGHEST)
    lr = jnp.where(alpha >= 0.0, alpha, 0.2 * alpha)
    g = jnp.maximum(jnp.max(lr, axis=0, keepdims=True), 0.0)
    e = jnp.exp(lr - g)
    ew = jnp.dot(e, s_ref[...],
                 preferred_element_type=jnp.float32,
                 precision=lax.Precision.HIGHEST)
    o_ref[...] = jnp.concatenate(
        [y * ew, e, jnp.zeros((y.shape[0], 8), jnp.float32)], axis=1)


def _k3(acc, A, S):
    return pl.pallas_call(
        _k3_body,
        out_shape=jax.ShapeDtypeStruct((MP, WIDTH), jnp.float32),
    )(acc, A, S)


def _k5_body(acc_ref, s_ref, o_ref):
    tot = acc_ref[0] + acc_ref[1]
    numer = tot[:, :128]
    den = jnp.dot(tot[:, 128:136], s_ref[...],
                  preferred_element_type=jnp.float32,
                  precision=lax.Precision.HIGHEST) + 1e-12
    r = numer / den
    o_ref[...] = jnp.where(r > 0.0, r, jnp.exp(jnp.minimum(r, 0.0)) - 1.0)


def _k5(acc, S):
    bn = 1000
    return pl.pallas_call(
        _k5_body,
        grid=(NP_PAD // bn,),
        in_specs=[
            pl.BlockSpec((2, bn, WIDTH), lambda i: (0, i, 0)),
            pl.BlockSpec((8, 128), lambda i: (0, 0)),
        ],
        out_specs=pl.BlockSpec((bn, 128), lambda i: (i, 0)),
        out_shape=jax.ShapeDtypeStruct((NP_PAD, 128), jnp.float32),
    )(acc, S)


def _sc_pass(table, idx, zeros, n_dst_pad, ch):
    n_chunks = idx.shape[1]
    rpt = n_dst_pad // 16
    mesh = plsc.VectorSubcoreMesh(core_axis_name="c", subcore_axis_name="s")

    @functools.partial(
        pl.kernel, mesh=mesh,
        compiler_params=pltpu.CompilerParams(use_tc_tiling_on_sc=False),
        out_type=jax.ShapeDtypeStruct((2, n_dst_pad, WIDTH), jnp.float32),
        scratch_types=[
            pltpu.VMEM((4, 2, ch), jnp.int32),
            pltpu.VMEM((3, ch, WIDTH), jnp.float32),
            pltpu.VMEM_SHARED((n_dst_pad, WIDTH), jnp.float32),
            pltpu.SemaphoreType.DMA,
            pltpu.SemaphoreType.DMA,
            pltpu.SemaphoreType.DMA,
        ],
    )
    def k(table_hbm, idx_hbm, z_hbm, out_hbm,
          ibuf, rows, acc, sem_i, sem_g, sem_s):
        c = lax.axis_index("c")
        s = lax.axis_index("s")
        wid = s * 2 + c
        for j in range(3):
            pltpu.async_copy(idx_hbm.at[wid, j], ibuf.at[j], sem_i)
        pltpu.sync_copy(z_hbm.at[pl.ds(s * rpt, rpt)],
                        acc.at[pl.ds(s * rpt, rpt)])
        plsc.subcore_barrier()

        def wait_idx():
            pltpu.make_async_copy(idx_hbm.at[wid, 0], ibuf.at[0],
                                  sem_i).wait()

        wait_idx()
        pltpu.async_copy(table_hbm.at[ibuf.at[0, 0]], rows.at[0], sem_g)
        wait_idx()
        pltpu.async_copy(table_hbm.at[ibuf.at[1, 0]], rows.at[1], sem_g)

        def body(i, carry):
            b = lax.rem(i, 3)
            bi = lax.rem(i, 4)
            pltpu.make_async_copy(table_hbm.at[ibuf.at[bi, 0]],
                                  rows.at[b], sem_g).wait()

            @pl.when(i >= 1)
            def _():
                pltpu.make_async_copy(rows.at[b], acc.at[ibuf.at[bi, 1]],
                                      sem_s).wait()

            pltpu.async_copy(rows.at[b], acc.at[ibuf.at[bi, 1]], sem_s,
                             add=True)

            @pl.when(i + 2 < n_chunks)
            def _():
                wait_idx()
                pltpu.async_copy(table_hbm.at[ibuf.at[lax.rem(i + 2, 4), 0]],
                                 rows.at[lax.rem(i + 2, 3)], sem_g)

            @pl.when(i + 3 < n_chunks)
            def _():
                pltpu.async_copy(idx_hbm.at[wid, i + 3],
                                 ibuf.at[lax.rem(i + 3, 4)], sem_i)

            return carry

        lax.fori_loop(0, n_chunks, body, 0)
        pltpu.make_async_copy(rows.at[0], acc.at[ibuf.at[0, 1]], sem_s).wait()
        plsc.subcore_barrier()
        pltpu.sync_copy(acc.at[pl.ds(s * rpt, rpt)],
                        out_hbm.at[c, pl.ds(s * rpt, rpt)])

    return k(table, idx, zeros)


def kernel(X, v_idx, e_idx, theta_w, theta_b, atten_e_w):
    n, in_c = X.shape
    h, _, oc = theta_w.shape
    hid = h * oc
    w_all = jnp.transpose(theta_w, (1, 0, 2)).reshape(in_c, hid)
    b_all = theta_b.reshape(1, hid)
    flat = atten_e_w.reshape(hid)
    mask = (jnp.arange(hid)[:, None] // oc
            == jnp.arange(8)[None, :]).astype(jnp.float32)
    a_mat = flat[:, None] * mask
    s_mat = mask.T
    p = v_idx.shape[0]
    nc = p // (32 * CH)
    vf = v_idx.astype(jnp.int32).reshape(32 * nc, CH)
    ef = e_idx.astype(jnp.int32).reshape(32 * nc, CH)

    t1, z_m, z_n, ve, ev = _k1(X, w_all, b_all, vf, ef)
    idx_ve = ve.reshape(32, nc, 2, CH)
    idx_ev = ev.reshape(32, nc, 2, CH)
    acc1 = _sc_pass(t1, idx_ve, z_m, MP, CH)
    t2 = _k3(acc1, a_mat, s_mat)
    acc2 = _sc_pass(t2, idx_ev, z_n, NP_PAD, CH)
    out = _k5(acc2, s_mat)
    return out[:n]


# speedup vs baseline: 95.2296x; 1.1173x over previous
import functools

import jax
import jax.numpy as jnp
from jax import lax
from jax.experimental import pallas as pl
from jax.experimental.pallas import tpu as pltpu
from jax.experimental.pallas import tpu_sc as plsc

N_EDGES = 5000
W1 = 128
W2 = 144
MP = 5040
NP_PAD = 10000
CH = 80


def _k0_body(v_ref, e_ref, zm_ref, zc_ref, zn_ref, ve_ref, ev_ref):
    zm_ref[...] = jnp.zeros_like(zm_ref)
    zc_ref[...] = jnp.zeros_like(zc_ref)
    zn_ref[...] = jnp.zeros_like(zn_ref)
    v = v_ref[...]
    e = e_ref[...]
    ve_ref[...] = jnp.concatenate([v[:, None, :], e[:, None, :]], axis=1)
    ev_ref[...] = jnp.concatenate([e[:, None, :], v[:, None, :]], axis=1)


def _k0(vf, ef):
    nr = vf.shape[0]
    return pl.pallas_call(
        _k0_body,
        out_shape=[
            jax.ShapeDtypeStruct((MP, W1), jnp.float32),
            jax.ShapeDtypeStruct((MP, 16), jnp.float32),
            jax.ShapeDtypeStruct((NP_PAD, W2), jnp.float32),
            jax.ShapeDtypeStruct((nr, 2, CH), jnp.int32),
            jax.ShapeDtypeStruct((nr, 2, CH), jnp.int32),
        ],
    )(vf, ef)


def _k3_body(a_ref, c_ref, w_ref, b_ref, am_ref, s_ref, o_ref):
    tot = a_ref[0] + a_ref[1]
    cnt16 = c_ref[0] + c_ref[1]
    cnt = cnt16[:, 0:1]
    mean_x = tot / jnp.maximum(cnt, 1.0)
    y = jnp.dot(mean_x, w_ref[...],
                preferred_element_type=jnp.float32,
                precision=lax.Precision.HIGHEST) + b_ref[...]
    y = jnp.where(cnt > 0.5, y, 0.0)
    alpha = jnp.dot(y, am_ref[...],
                    preferred_element_type=jnp.float32,
                    precision=lax.Precision.HIGHEST)
    lr = jnp.where(alpha >= 0.0, alpha, 0.2 * alpha)
    g = jnp.maximum(jnp.max(lr, axis=0, keepdims=True), 0.0)
    e = jnp.exp(lr - g)
    ew = jnp.dot(e, s_ref[...],
                 preferred_element_type=jnp.float32,
                 precision=lax.Precision.HIGHEST)
    o_ref[...] = jnp.concatenate(
        [y * ew, e, jnp.zeros((y.shape[0], 8), jnp.float32)], axis=1)


def _k3(acc, cnt, W, b, A, S):
    return pl.pallas_call(
        _k3_body,
        out_shape=jax.ShapeDtypeStruct((MP, W2), jnp.float32),
    )(acc, cnt, W, b, A, S)


def _k5_body(acc_ref, s_ref, o_ref):
    tot = acc_ref[0] + acc_ref[1]
    numer = tot[:, :128]
    den = jnp.dot(tot[:, 128:136], s_ref[...],
                  preferred_element_type=jnp.float32,
                  precision=lax.Precision.HIGHEST) + 1e-12
    r = numer / den
    o_ref[...] = jnp.where(r > 0.0, r, jnp.exp(jnp.minimum(r, 0.0)) - 1.0)


def _k5(acc, S):
    bn = 1000
    return pl.pallas_call(
        _k5_body,
        grid=(NP_PAD // bn,),
        in_specs=[
            pl.BlockSpec((2, bn, W2), lambda i: (0, i, 0)),
            pl.BlockSpec((8, 128), lambda i: (0, 0)),
        ],
        out_specs=pl.BlockSpec((bn, 128), lambda i: (i, 0)),
        out_shape=jax.ShapeDtypeStruct((NP_PAD, 128), jnp.float32),
    )(acc, S)


_MESH = plsc.VectorSubcoreMesh(core_axis_name="c", subcore_axis_name="s")
_PARAMS = pltpu.CompilerParams(use_tc_tiling_on_sc=False)


def _sc_pass1(table, idx, zacc, zcnt):
    n_chunks = idx.shape[1]
    rpt = MP // 16

    @functools.partial(
        pl.kernel, mesh=_MESH, compiler_params=_PARAMS,
        out_type=[
            jax.ShapeDtypeStruct((2, MP, W1), jnp.float32),
            jax.ShapeDtypeStruct((2, MP, 16), jnp.float32),
        ],
        scratch_types=[
            pltpu.VMEM((4, 2, CH), jnp.int32),
            pltpu.VMEM((3, CH, W1), jnp.float32),
            pltpu.VMEM((CH, 16), jnp.float32),
            pltpu.VMEM_SHARED((MP, W1), jnp.float32),
            pltpu.VMEM_SHARED((MP, 16), jnp.float32),
            pltpu.SemaphoreType.DMA,
            pltpu.SemaphoreType.DMA,
            pltpu.SemaphoreType.DMA,
        ],
    )
    def k(table_hbm, idx_hbm, z_hbm, zc_hbm, acc_out, cnt_out,
          ibuf, rows, ones, acc, cnt, sem_i, sem_g, sem_s):
        c = lax.axis_index("c")
        s = lax.axis_index("s")
        wid = s * 2 + c
        for j in range(3):
            pltpu.async_copy(idx_hbm.at[wid, j], ibuf.at[j], sem_i)

        def fill(r, carry):
            ones[r] = jnp.full((16,), 1.0, jnp.float32)
            return carry

        lax.fori_loop(0, CH, fill, 0)
        pltpu.sync_copy(z_hbm.at[pl.ds(s * rpt, rpt)],
                        acc.at[pl.ds(s * rpt, rpt)])
        pltpu.sync_copy(zc_hbm.at[pl.ds(s * rpt, rpt)],
                        cnt.at[pl.ds(s * rpt, rpt)])
        plsc.subcore_barrier()

        def wait_idx():
            pltpu.make_async_copy(idx_hbm.at[wid, 0], ibuf.at[0],
                                  sem_i).wait()

        wait_idx()
        pltpu.async_copy(table_hbm.at[ibuf.at[0, 0]], rows.at[0], sem_g)
        wait_idx()
        pltpu.async_copy(table_hbm.at[ibuf.at[1, 0]], rows.at[1], sem_g)

        def body(i, carry):
            b = lax.rem(i, 3)
            bi = lax.rem(i, 4)
            pltpu.make_async_copy(table_hbm.at[ibuf.at[bi, 0]],
                                  rows.at[b], sem_g).wait()

            @pl.when(i >= 1)
            def _():
                pltpu.make_async_copy(rows.at[b], acc.at[ibuf.at[bi, 1]],
                                      sem_s).wait()
                pltpu.make_async_copy(ones, cnt.at[ibuf.at[bi, 1]],
                                      sem_s).wait()

            pltpu.async_copy(rows.at[b], acc.at[ibuf.at[bi, 1]], sem_s,
                             add=True)
            pltpu.async_copy(ones, cnt.at[ibuf.at[bi, 1]], sem_s, add=True)

            @pl.when(i + 2 < n_chunks)
            def _():
                wait_idx()
                pltpu.async_copy(table_hbm.at[ibuf.at[lax.rem(i + 2, 4), 0]],
                                 rows.at[lax.rem(i + 2, 3)], sem_g)

            @pl.when(i + 3 < n_chunks)
            def _():
                pltpu.async_copy(idx_hbm.at[wid, i + 3],
                                 ibuf.at[lax.rem(i + 3, 4)], sem_i)

            return carry

        lax.fori_loop(0, n_chunks, body, 0)
        pltpu.make_async_copy(rows.at[0], acc.at[ibuf.at[0, 1]], sem_s).wait()
        pltpu.make_async_copy(ones, cnt.at[ibuf.at[0, 1]], sem_s).wait()
        plsc.subcore_barrier()
        pltpu.sync_copy(acc.at[pl.ds(s * rpt, rpt)],
                        acc_out.at[c, pl.ds(s * rpt, rpt)])
        pltpu.sync_copy(cnt.at[pl.ds(s * rpt, rpt)],
                        cnt_out.at[c, pl.ds(s * rpt, rpt)])

    return k(table, idx, zacc, zcnt)


def _sc_pass2(table, idx, zacc):
    n_chunks = idx.shape[1]
    rpt = NP_PAD // 16

    @functools.partial(
        pl.kernel, mesh=_MESH, compiler_params=_PARAMS,
        out_type=jax.ShapeDtypeStruct((2, NP_PAD, W2), jnp.float32),
        scratch_types=[
            pltpu.VMEM((4, 2, CH), jnp.int32),
            pltpu.VMEM((3, CH, W2), jnp.float32),
            pltpu.VMEM_SHARED((NP_PAD, W2), jnp.float32),
            pltpu.SemaphoreType.DMA,
            pltpu.SemaphoreType.DMA,
            pltpu.SemaphoreType.DMA,
        ],
    )
    def k(table_hbm, idx_hbm, z_hbm, out_hbm,
          ibuf, rows, acc, sem_i, sem_g, sem_s):
        c = lax.axis_index("c")
        s = lax.axis_index("s")
        wid = s * 2 + c
        for j in range(3):
            pltpu.async_copy(idx_hbm.at[wid, j], ibuf.at[j], sem_i)
        pltpu.sync_copy(z_hbm.at[pl.ds(s * rpt, rpt)],
                        acc.at[pl.ds(s * rpt, rpt)])
        plsc.subcore_barrier()

        def wait_idx():
            pltpu.make_async_copy(idx_hbm.at[wid, 0], ibuf.at[0],
                                  sem_i).wait()

        wait_idx()
        pltpu.async_copy(table_hbm.at[ibuf.at[0, 0]], rows.at[0], sem_g)
        wait_idx()
        pltpu.async_copy(table_hbm.at[ibuf.at[1, 0]], rows.at[1], sem_g)

        def body(i, carry):
            b = lax.rem(i, 3)
            bi = lax.rem(i, 4)
            pltpu.make_async_copy(table_hbm.at[ibuf.at[bi, 0]],
                                  rows.at[b], sem_g).wait()

            @pl.when(i >= 1)
            def _():
                pltpu.make_async_copy(rows.at[b], acc.at[ibuf.at[bi, 1]],
                                      sem_s).wait()

            pltpu.async_copy(rows.at[b], acc.at[ibuf.at[bi, 1]], sem_s,
                             add=True)

            @pl.when(i + 2 < n_chunks)
            def _():
                wait_idx()
                pltpu.async_copy(table_hbm.at[ibuf.at[lax.rem(i + 2, 4), 0]],
                                 rows.at[lax.rem(i + 2, 3)], sem_g)

            @pl.when(i + 3 < n_chunks)
            def _():
                pltpu.async_copy(idx_hbm.at[wid, i + 3],
                                 ibuf.at[lax.rem(i + 3, 4)], sem_i)

            return carry

        lax.fori_loop(0, n_chunks, body, 0)
        pltpu.make_async_copy(rows.at[0], acc.at[ibuf.at[0, 1]], sem_s).wait()
        plsc.subcore_barrier()
        pltpu.sync_copy(acc.at[pl.ds(s * rpt, rpt)],
                        out_hbm.at[c, pl.ds(s * rpt, rpt)])

    return k(table, idx, zacc)


def kernel(X, v_idx, e_idx, theta_w, theta_b, atten_e_w):
    n, in_c = X.shape
    h, _, oc = theta_w.shape
    hid = h * oc
    w_all = jnp.transpose(theta_w, (1, 0, 2)).reshape(in_c, hid)
    b_all = theta_b.reshape(1, hid)
    flat = atten_e_w.reshape(hid)
    mask = (jnp.arange(hid)[:, None] // oc
            == jnp.arange(8)[None, :]).astype(jnp.float32)
    a_mat = flat[:, None] * mask
    s_mat = mask.T
    p = v_idx.shape[0]
    nc = p // (32 * CH)
    vf = v_idx.astype(jnp.int32).reshape(32 * nc, CH)
    ef = e_idx.astype(jnp.int32).reshape(32 * nc, CH)

    z_m, z_c, z_n, ve, ev = _k0(vf, ef)
    idx_ve = ve.reshape(32, nc, 2, CH)
    idx_ev = ev.reshape(32, nc, 2, CH)
    acc1, cnt1 = _sc_pass1(X, idx_ve, z_m, z_c)
    t2 = _k3(acc1, cnt1, w_all, b_all, a_mat, s_mat)
    acc2 = _sc_pass2(t2, idx_ev, z_n)
    out = _k5(acc2, s_mat)
    return out[:n]
